# initial kernel scaffold (unmeasured)
import jax
import jax.numpy as jnp
from jax import lax
from jax.experimental import pallas as pl
from jax.experimental.pallas import tpu as pltpu

N_Z = 4
M = 4096
D = 4096
CH = M // N_Z
SUB = 128
NSUB = CH // SUB


def kernel(partial, resid, gamma):
    x = partial[0].astype(jnp.bfloat16)
    r = resid.astype(jnp.bfloat16)
    g = gamma.reshape(1, D)

    def body(x_ref, r_ref, g_ref, out_ref, comm, loc, rbuf, ybuf,
             rs_send, rs_recv, ag_send, ag_recv, ldma, sdma):
        my_x = lax.axis_index("x")
        my_y = lax.axis_index("y")
        my_z = lax.axis_index("z")
        nxt = (my_z + 1) % N_Z
        prv = (my_z - 1) % N_Z

        barrier = pltpu.get_barrier_semaphore()
        for nbr in (prv, nxt):
            pl.semaphore_signal(
                barrier, inc=1,
                device_id=(my_x, my_y, nbr),
                device_id_type=pl.DeviceIdType.MESH,
            )
        pl.semaphore_wait(barrier, 2)

        def load_x(c, dst):
            cp = pltpu.make_async_copy(x_ref.at[pl.ds(c * CH, CH), :], dst, ldma)
            cp.start()
            return cp

        load_x(my_z, comm.at[0]).wait()

        for s in range(N_Z - 1):
            rdma = pltpu.make_async_remote_copy(
                src_ref=comm.at[s],
                dst_ref=comm.at[s + 1],
                send_sem=rs_send.at[s],
                recv_sem=rs_recv.at[s],
                device_id=(my_x, my_y, nxt),
                device_id_type=pl.DeviceIdType.MESH,
            )
            rdma.start()
            c_in = (my_z - s - 1) % N_Z
            lcp = load_x(c_in, loc)
            lcp.wait()
            rdma.wait()
            comm[s + 1, :, :] = comm[s + 1, :, :] + loc[:, :]

        cmine = (my_z + 1) % N_Z

        rcp = pltpu.make_async_copy(r_ref.at[pl.ds(cmine * CH, CH), :], rbuf, ldma)
        rcp.start()
        rcp.wait()

        for h in range(NSUB):
            y = (comm[3, h * SUB:(h + 1) * SUB, :].astype(jnp.float32)
                 + rbuf[h * SUB:(h + 1) * SUB, :].astype(jnp.float32))
            ms = jnp.mean(y * y, axis=1, keepdims=True) + 1e-6
            o = y * lax.rsqrt(ms) * g_ref[:, :]
            ybuf[:, :] = o
            comm[0, h * SUB:(h + 1) * SUB, :] = o.astype(jnp.bfloat16)
            st = pltpu.make_async_copy(
                ybuf, out_ref.at[pl.ds(cmine * CH + h * SUB, SUB), :], sdma)
            st.start()
            st.wait()

        for s in range(N_Z - 1):
            src_slot = 0 if s == 0 else s
            dst_slot = s + 1
            rdma = pltpu.make_async_remote_copy(
                src_ref=comm.at[src_slot],
                dst_ref=comm.at[dst_slot],
                send_sem=ag_send.at[s],
                recv_sem=ag_recv.at[s],
                device_id=(my_x, my_y, nxt),
                device_id_type=pl.DeviceIdType.MESH,
            )
            rdma.start()
            rdma.wait()
            c_got = (my_z - s) % N_Z
            for h in range(NSUB):
                ybuf[:, :] = comm[
                    dst_slot, h * SUB:(h + 1) * SUB, :].astype(jnp.float32)
                st = pltpu.make_async_copy(
                    ybuf, out_ref.at[pl.ds(c_got * CH + h * SUB, SUB), :], sdma)
                st.start()
                st.wait()

    return pl.pallas_call(
        body,
        out_shape=jax.ShapeDtypeStruct((M, D), jnp.float32),
        in_specs=[
            pl.BlockSpec(memory_space=pltpu.ANY),
            pl.BlockSpec(memory_space=pltpu.ANY),
            pl.BlockSpec(memory_space=pltpu.VMEM),
        ],
        out_specs=pl.BlockSpec(memory_space=pltpu.ANY),
        scratch_shapes=[
            pltpu.VMEM((N_Z, CH, D), jnp.bfloat16),
            pltpu.VMEM((CH, D), jnp.bfloat16),
            pltpu.VMEM((CH, D), jnp.bfloat16),
            pltpu.VMEM((SUB, D), jnp.float32),
            pltpu.SemaphoreType.DMA((N_Z - 1,)),
            pltpu.SemaphoreType.DMA((N_Z - 1,)),
            pltpu.SemaphoreType.DMA((N_Z - 1,)),
            pltpu.SemaphoreType.DMA((N_Z - 1,)),
            pltpu.SemaphoreType.DMA,
            pltpu.SemaphoreType.DMA,
        ],
        compiler_params=pltpu.CompilerParams(
            collective_id=0,
            vmem_limit_bytes=100 * 1024 * 1024,
        ),
    )(x, r, g)


# baseline (device time: 728470 ns/iter reference)
import jax
import jax.numpy as jnp
from jax import lax
from jax.experimental import pallas as pl
from jax.experimental.pallas import tpu as pltpu

N_Z = 4
M = 4096
D = 4096
CH = M // N_Z
SUB = 128
NSUB = CH // SUB


def kernel(partial, resid, gamma):
    x = partial[0].astype(jnp.bfloat16)
    r = resid.astype(jnp.bfloat16)
    g = gamma.reshape(1, D)

    def body(x_ref, r_ref, g_ref, out_ref, comm, loc, rbuf, ybuf,
             rs_send, rs_recv, ag_send, ag_recv, ldma, sdma):
        my_x = lax.axis_index("x")
        my_y = lax.axis_index("y")
        my_z = lax.axis_index("z")
        nxt = (my_z + 1) % N_Z
        prv = (my_z - 1) % N_Z

        barrier = pltpu.get_barrier_semaphore()
        for nbr in (prv, nxt):
            pl.semaphore_signal(
                barrier, inc=1,
                device_id=(my_x, my_y, nbr),
                device_id_type=pl.DeviceIdType.MESH,
            )
        pl.semaphore_wait(barrier, 2)

        def load_x(c, dst):
            cp = pltpu.make_async_copy(x_ref.at[pl.ds(c * CH, CH), :], dst, ldma)
            cp.start()
            return cp

        load_x(my_z, comm.at[0]).wait()

        for s in range(N_Z - 1):
            rdma = pltpu.make_async_remote_copy(
                src_ref=comm.at[s],
                dst_ref=comm.at[s + 1],
                send_sem=rs_send.at[s],
                recv_sem=rs_recv.at[s],
                device_id=(my_x, my_y, nxt),
                device_id_type=pl.DeviceIdType.MESH,
            )
            rdma.start()
            c_in = (my_z - s - 1) % N_Z
            lcp = load_x(c_in, loc)
            lcp.wait()
            rdma.wait()
            comm[s + 1, :, :] = comm[s + 1, :, :] + loc[:, :]

        cmine = (my_z + 1) % N_Z

        rcp = pltpu.make_async_copy(r_ref.at[pl.ds(cmine * CH, CH), :], rbuf, ldma)
        rcp.start()
        rcp.wait()

        for h in range(NSUB):
            y = (comm[3, h * SUB:(h + 1) * SUB, :].astype(jnp.float32)
                 + rbuf[h * SUB:(h + 1) * SUB, :].astype(jnp.float32))
            ms = jnp.mean(y * y, axis=1, keepdims=True) + 1e-6
            o = y * lax.rsqrt(ms) * g_ref[:, :]
            ybuf[:, :] = o
            comm[0, h * SUB:(h + 1) * SUB, :] = o.astype(jnp.bfloat16)
            st = pltpu.make_async_copy(
                ybuf, out_ref.at[pl.ds(cmine * CH + h * SUB, SUB), :], sdma)
            st.start()
            st.wait()

        for s in range(N_Z - 1):
            src_slot = 0 if s == 0 else s
            dst_slot = s + 1
            rdma = pltpu.make_async_remote_copy(
                src_ref=comm.at[src_slot],
                dst_ref=comm.at[dst_slot],
                send_sem=ag_send.at[s],
                recv_sem=ag_recv.at[s],
                device_id=(my_x, my_y, nxt),
                device_id_type=pl.DeviceIdType.MESH,
            )
            rdma.start()
            rdma.wait()
            c_got = (my_z - s) % N_Z
            for h in range(NSUB):
                ybuf[:, :] = comm[
                    dst_slot, h * SUB:(h + 1) * SUB, :].astype(jnp.float32)
                st = pltpu.make_async_copy(
                    ybuf, out_ref.at[pl.ds(c_got * CH + h * SUB, SUB), :], sdma)
                st.start()
                st.wait()

    return pl.pallas_call(
        body,
        out_shape=jax.ShapeDtypeStruct((M, D), jnp.float32),
        in_specs=[
            pl.BlockSpec(memory_space=pl.ANY),
            pl.BlockSpec(memory_space=pl.ANY),
            pl.BlockSpec(memory_space=pltpu.VMEM),
        ],
        out_specs=pl.BlockSpec(memory_space=pl.ANY),
        scratch_shapes=[
            pltpu.VMEM((N_Z, CH, D), jnp.bfloat16),
            pltpu.VMEM((CH, D), jnp.bfloat16),
            pltpu.VMEM((CH, D), jnp.bfloat16),
            pltpu.VMEM((SUB, D), jnp.float32),
            pltpu.SemaphoreType.DMA((N_Z - 1,)),
            pltpu.SemaphoreType.DMA((N_Z - 1,)),
            pltpu.SemaphoreType.DMA((N_Z - 1,)),
            pltpu.SemaphoreType.DMA((N_Z - 1,)),
            pltpu.SemaphoreType.DMA,
            pltpu.SemaphoreType.DMA,
        ],
        compiler_params=pltpu.CompilerParams(
            collective_id=0,
            vmem_limit_bytes=100 * 1024 * 1024,
        ),
    )(x, r, g)


# device time: 413573 ns/iter; 1.7614x vs baseline; 1.7614x over previous
import jax
import jax.numpy as jnp
from jax import lax
from jax.experimental import pallas as pl
from jax.experimental.pallas import tpu as pltpu

N_Z = 4
M = 4096
D = 4096
COL = M // 4
B = COL // N_Z
HB = COL // 2
SUB = 128
EPS = 1e-6


def kernel(partial, resid, gamma):
    x = partial[0].astype(jnp.bfloat16)
    r = resid.astype(jnp.bfloat16)
    g = gamma.reshape(1, D)

    def body(x_ref, r_ref, g_ref, out_ref,
             commz, locb, rbuf, colbuf, gx, gy, gd, ybuf,
             rs_send, rs_recv, ag_send, ag_recv,
             xy_send, xy_recv, ldma, rsem, sdma):
        my_x = lax.axis_index("x")
        my_y = lax.axis_index("y")
        my_z = lax.axis_index("z")
        nxt = (my_z + 1) % N_Z
        prv = (my_z - 1) % N_Z
        q = 2 * my_x + my_y
        col0 = q * COL
        cmine = (my_z + 1) % N_Z
        xn = (1 - my_x, my_y, my_z)
        yn = (my_x, 1 - my_y, my_z)

        barrier = pltpu.get_barrier_semaphore()
        for dev in ((my_x, my_y, prv), (my_x, my_y, nxt), xn, yn):
            pl.semaphore_signal(
                barrier, inc=1, device_id=dev,
                device_id_type=pl.DeviceIdType.MESH,
            )
        pl.semaphore_wait(barrier, 4)

        def load_x(c, dst):
            cp = pltpu.make_async_copy(
                x_ref.at[pl.ds(col0 + c * B, B), :], dst, ldma)
            cp.start()
            return cp

        load_x(my_z, commz.at[0]).wait()
        rcp = pltpu.make_async_copy(
            r_ref.at[pl.ds(col0 + cmine * B, B), :], rbuf, rsem)
        rcp.start()

        for s in range(N_Z - 1):
            rdma = pltpu.make_async_remote_copy(
                src_ref=commz.at[s], dst_ref=commz.at[s + 1],
                send_sem=rs_send.at[s], recv_sem=rs_recv.at[s],
                device_id=(my_x, my_y, nxt),
                device_id_type=pl.DeviceIdType.MESH,
            )
            rdma.start()
            load_x((my_z - s - 1) % N_Z, locb).wait()
            rdma.wait()
            commz[s + 1, :, :] = commz[s + 1, :, :] + locb[:, :]

        pending = [None, None]
        ctr = [0]

        def emit_store(val_f32, row0):
            slot = ctr[0] & 1
            if pending[slot] is not None:
                pending[slot].wait()
            ybuf[slot, :, :] = val_f32
            cp = pltpu.make_async_copy(
                ybuf.at[slot], out_ref.at[pl.ds(row0, SUB), :], sdma.at[slot])
            cp.start()
            pending[slot] = cp
            ctr[0] += 1

        def store_bf16_rows(buf, off, n_rows, out_row0):
            for h in range(n_rows // SUB):
                emit_store(
                    buf[pl.ds(off + h * SUB, SUB), :].astype(jnp.float32),
                    out_row0 + h * SUB)

        rcp.wait()
        for h in range(B // SUB):
            y = (commz[3, h * SUB:(h + 1) * SUB, :].astype(jnp.float32)
                 + rbuf[h * SUB:(h + 1) * SUB, :].astype(jnp.float32))
            ms = jnp.mean(y * y, axis=1, keepdims=True) + EPS
            o = y * lax.rsqrt(ms) * g_ref[:, :]
            colbuf[pl.ds(cmine * B + h * SUB, SUB), :] = o.astype(jnp.bfloat16)
            emit_store(o, col0 + cmine * B + h * SUB)

        for s in range(N_Z - 1):
            bs = (my_z - s + 1) % N_Z
            off = bs * B
            rdma = pltpu.make_async_remote_copy(
                src_ref=colbuf.at[pl.ds(off, B), :],
                dst_ref=colbuf.at[pl.ds(off, B), :],
                send_sem=ag_send.at[s], recv_sem=ag_recv.at[s],
                device_id=(my_x, my_y, nxt),
                device_id_type=pl.DeviceIdType.MESH,
            )
            rdma.start()
            if s > 0:
                store_bf16_rows(colbuf, off, B, col0 + off)
            rdma.wait()
        last = (my_z - 2) % N_Z
        store_bf16_rows(colbuf, last * B, B, col0 + last * B)

        qx = 2 * (1 - my_x) + my_y
        qy = 2 * my_x + (1 - my_y)
        qd = 2 * (1 - my_x) + (1 - my_y)

        op1 = pltpu.make_async_remote_copy(
            src_ref=colbuf, dst_ref=gx,
            send_sem=xy_send.at[0], recv_sem=xy_recv.at[0],
            device_id=xn, device_id_type=pl.DeviceIdType.MESH)
        op1.start()
        op2 = pltpu.make_async_remote_copy(
            src_ref=colbuf, dst_ref=gy,
            send_sem=xy_send.at[1], recv_sem=xy_recv.at[1],
            device_id=yn, device_id_type=pl.DeviceIdType.MESH)
        op2.start()

        op1.wait()
        op3 = pltpu.make_async_remote_copy(
            src_ref=gx.at[pl.ds(0, HB), :], dst_ref=gd.at[pl.ds(0, HB), :],
            send_sem=xy_send.at[2], recv_sem=xy_recv.at[2],
            device_id=yn, device_id_type=pl.DeviceIdType.MESH)
        op3.start()
        op2.wait()
        op4 = pltpu.make_async_remote_copy(
            src_ref=gy.at[pl.ds(HB, HB), :], dst_ref=gd.at[pl.ds(HB, HB), :],
            send_sem=xy_send.at[3], recv_sem=xy_recv.at[3],
            device_id=xn, device_id_type=pl.DeviceIdType.MESH)
        op4.start()

        store_bf16_rows(gx, 0, COL, qx * COL)
        store_bf16_rows(gy, 0, COL, qy * COL)

        op3.wait()
        op4.wait()
        store_bf16_rows(gd, 0, COL, qd * COL)

        for cp in pending:
            if cp is not None:
                cp.wait()

    return pl.pallas_call(
        body,
        out_shape=jax.ShapeDtypeStruct((M, D), jnp.float32),
        in_specs=[
            pl.BlockSpec(memory_space=pl.ANY),
            pl.BlockSpec(memory_space=pl.ANY),
            pl.BlockSpec(memory_space=pltpu.VMEM),
        ],
        out_specs=pl.BlockSpec(memory_space=pl.ANY),
        scratch_shapes=[
            pltpu.VMEM((N_Z, B, D), jnp.bfloat16),
            pltpu.VMEM((B, D), jnp.bfloat16),
            pltpu.VMEM((B, D), jnp.bfloat16),
            pltpu.VMEM((COL, D), jnp.bfloat16),
            pltpu.VMEM((COL, D), jnp.bfloat16),
            pltpu.VMEM((COL, D), jnp.bfloat16),
            pltpu.VMEM((COL, D), jnp.bfloat16),
            pltpu.VMEM((2, SUB, D), jnp.float32),
            pltpu.SemaphoreType.DMA((N_Z - 1,)),
            pltpu.SemaphoreType.DMA((N_Z - 1,)),
            pltpu.SemaphoreType.DMA((N_Z - 1,)),
            pltpu.SemaphoreType.DMA((N_Z - 1,)),
            pltpu.SemaphoreType.DMA((4,)),
            pltpu.SemaphoreType.DMA((4,)),
            pltpu.SemaphoreType.DMA,
            pltpu.SemaphoreType.DMA,
            pltpu.SemaphoreType.DMA((2,)),
        ],
        compiler_params=pltpu.CompilerParams(
            collective_id=0,
            vmem_limit_bytes=100 * 1024 * 1024,
        ),
    )(x, r, g)


# device time: 346045 ns/iter; 2.1051x vs baseline; 1.1951x over previous
import jax
import jax.numpy as jnp
from jax import lax
from jax.experimental import pallas as pl
from jax.experimental.pallas import tpu as pltpu

N_Z = 4
M = 4096
D = 4096
COL = M // 4
B = COL // N_Z
HB = COL // 2
SUB = 128
EPS = 1e-6


def kernel(partial, resid, gamma):
    x = partial[0]
    r = resid
    g = gamma.reshape(1, D)

    def body(x_ref, r_ref, g_ref, out_ref,
             commz, locb, rbuf, colbuf, gx, gy, gd, ybuf,
             rs_send, rs_recv, ag_send, ag_recv,
             xy_send, xy_recv, ldma, rsem, sdma):
        my_x = lax.axis_index("x")
        my_y = lax.axis_index("y")
        my_z = lax.axis_index("z")
        nxt = (my_z + 1) % N_Z
        prv = (my_z - 1) % N_Z
        q = 2 * my_x + my_y
        col0 = q * COL
        cmine = (my_z + 1) % N_Z
        xn = (1 - my_x, my_y, my_z)
        yn = (my_x, 1 - my_y, my_z)

        barrier = pltpu.get_barrier_semaphore()
        for dev in ((my_x, my_y, prv), (my_x, my_y, nxt), xn, yn):
            pl.semaphore_signal(
                barrier, inc=1, device_id=dev,
                device_id_type=pl.DeviceIdType.MESH,
            )
        pl.semaphore_wait(barrier, 4)

        def load_x(c):
            cp = pltpu.make_async_copy(
                x_ref.at[pl.ds(col0 + c * B, B), :], locb, ldma)
            cp.start()
            return cp

        load_x(my_z).wait()
        commz[0, :, :] = locb[:, :].astype(jnp.bfloat16)
        rcp = pltpu.make_async_copy(
            r_ref.at[pl.ds(col0 + cmine * B, B), :], rbuf, rsem)
        rcp.start()

        for s in range(N_Z - 1):
            rdma = pltpu.make_async_remote_copy(
                src_ref=commz.at[s], dst_ref=commz.at[s + 1],
                send_sem=rs_send.at[s], recv_sem=rs_recv.at[s],
                device_id=(my_x, my_y, nxt),
                device_id_type=pl.DeviceIdType.MESH,
            )
            rdma.start()
            load_x((my_z - s - 1) % N_Z).wait()
            rdma.wait()
            commz[s + 1, :, :] = (commz[s + 1, :, :]
                                  + locb[:, :].astype(jnp.bfloat16))

        pending = [None, None]
        ctr = [0]

        def emit_store(val_f32, row0):
            slot = ctr[0] & 1
            if pending[slot] is not None:
                pending[slot].wait()
            ybuf[slot, :, :] = val_f32
            cp = pltpu.make_async_copy(
                ybuf.at[slot], out_ref.at[pl.ds(row0, SUB), :], sdma.at[slot])
            cp.start()
            pending[slot] = cp
            ctr[0] += 1

        def store_bf16_rows(buf, off, n_rows, out_row0):
            for h in range(n_rows // SUB):
                emit_store(
                    buf[pl.ds(off + h * SUB, SUB), :].astype(jnp.float32),
                    out_row0 + h * SUB)

        rcp.wait()
        for h in range(B // SUB):
            y = (commz[3, h * SUB:(h + 1) * SUB, :].astype(jnp.float32)
                 + rbuf[h * SUB:(h + 1) * SUB, :])
            ms = jnp.mean(y * y, axis=1, keepdims=True) + EPS
            o = y * lax.rsqrt(ms) * g_ref[:, :]
            colbuf[pl.ds(cmine * B + h * SUB, SUB), :] = o.astype(jnp.bfloat16)
            emit_store(o, col0 + cmine * B + h * SUB)

        for s in range(N_Z - 1):
            bs = (my_z - s + 1) % N_Z
            off = bs * B
            rdma = pltpu.make_async_remote_copy(
                src_ref=colbuf.at[pl.ds(off, B), :],
                dst_ref=colbuf.at[pl.ds(off, B), :],
                send_sem=ag_send.at[s], recv_sem=ag_recv.at[s],
                device_id=(my_x, my_y, nxt),
                device_id_type=pl.DeviceIdType.MESH,
            )
            rdma.start()
            if s > 0:
                store_bf16_rows(colbuf, off, B, col0 + off)
            rdma.wait()
        last = (my_z - 2) % N_Z
        store_bf16_rows(colbuf, last * B, B, col0 + last * B)

        qx = 2 * (1 - my_x) + my_y
        qy = 2 * my_x + (1 - my_y)
        qd = 2 * (1 - my_x) + (1 - my_y)

        op1 = pltpu.make_async_remote_copy(
            src_ref=colbuf, dst_ref=gx,
            send_sem=xy_send.at[0], recv_sem=xy_recv.at[0],
            device_id=xn, device_id_type=pl.DeviceIdType.MESH)
        op1.start()
        op2 = pltpu.make_async_remote_copy(
            src_ref=colbuf, dst_ref=gy,
            send_sem=xy_send.at[1], recv_sem=xy_recv.at[1],
            device_id=yn, device_id_type=pl.DeviceIdType.MESH)
        op2.start()

        op1.wait()
        op3 = pltpu.make_async_remote_copy(
            src_ref=gx.at[pl.ds(0, HB), :], dst_ref=gd.at[pl.ds(0, HB), :],
            send_sem=xy_send.at[2], recv_sem=xy_recv.at[2],
            device_id=yn, device_id_type=pl.DeviceIdType.MESH)
        op3.start()
        op2.wait()
        op4 = pltpu.make_async_remote_copy(
            src_ref=gy.at[pl.ds(HB, HB), :], dst_ref=gd.at[pl.ds(HB, HB), :],
            send_sem=xy_send.at[3], recv_sem=xy_recv.at[3],
            device_id=xn, device_id_type=pl.DeviceIdType.MESH)
        op4.start()

        store_bf16_rows(gx, 0, COL, qx * COL)
        store_bf16_rows(gy, 0, COL, qy * COL)

        op3.wait()
        op4.wait()
        store_bf16_rows(gd, 0, COL, qd * COL)

        for cp in pending:
            if cp is not None:
                cp.wait()

    return pl.pallas_call(
        body,
        out_shape=jax.ShapeDtypeStruct((M, D), jnp.float32),
        in_specs=[
            pl.BlockSpec(memory_space=pl.ANY),
            pl.BlockSpec(memory_space=pl.ANY),
            pl.BlockSpec(memory_space=pltpu.VMEM),
        ],
        out_specs=pl.BlockSpec(memory_space=pl.ANY),
        scratch_shapes=[
            pltpu.VMEM((N_Z, B, D), jnp.bfloat16),
            pltpu.VMEM((B, D), jnp.float32),
            pltpu.VMEM((B, D), jnp.float32),
            pltpu.VMEM((COL, D), jnp.bfloat16),
            pltpu.VMEM((COL, D), jnp.bfloat16),
            pltpu.VMEM((COL, D), jnp.bfloat16),
            pltpu.VMEM((COL, D), jnp.bfloat16),
            pltpu.VMEM((2, SUB, D), jnp.float32),
            pltpu.SemaphoreType.DMA((N_Z - 1,)),
            pltpu.SemaphoreType.DMA((N_Z - 1,)),
            pltpu.SemaphoreType.DMA((N_Z - 1,)),
            pltpu.SemaphoreType.DMA((N_Z - 1,)),
            pltpu.SemaphoreType.DMA((4,)),
            pltpu.SemaphoreType.DMA((4,)),
            pltpu.SemaphoreType.DMA,
            pltpu.SemaphoreType.DMA,
            pltpu.SemaphoreType.DMA((2,)),
        ],
        compiler_params=pltpu.CompilerParams(
            collective_id=0,
            vmem_limit_bytes=100 * 1024 * 1024,
        ),
    )(x, r, g)


# device time: 256211 ns/iter; 2.8432x vs baseline; 1.3506x over previous
import jax
import jax.numpy as jnp
from jax import lax
from jax.experimental import pallas as pl
from jax.experimental.pallas import tpu as pltpu

N_Z = 4
M = 4096
D = 4096
COL = M // 4
B = COL // N_Z
SUB = 128
EPS = 1e-6


def kernel(partial, resid, gamma):
    x = partial[0]
    r = resid
    g = gamma.reshape(1, D)

    def body(x_ref, r_ref, g_ref, out_ref,
             commz, locb, rbuf, colbuf, gx, gy, gd, ybuf,
             rs_send, rs_recv, rs2_send, rs2_recv, agz_send, agz_recv,
             xs, ys, gxr, gyr, fx, fy, gdr, ldma, rsem, sdma):
        my_x = lax.axis_index("x")
        my_y = lax.axis_index("y")
        my_z = lax.axis_index("z")
        nxt = (my_z + 1) % N_Z
        prv = (my_z - 1) % N_Z
        q = 2 * my_x + my_y
        col0 = q * COL
        cmine = (my_z + 1) % N_Z
        xn = (1 - my_x, my_y, my_z)
        yn = (my_x, 1 - my_y, my_z)

        barrier = pltpu.get_barrier_semaphore()
        for dev in ((my_x, my_y, prv), (my_x, my_y, nxt), xn, yn):
            pl.semaphore_signal(
                barrier, inc=1, device_id=dev,
                device_id_type=pl.DeviceIdType.MESH,
            )
        pl.semaphore_wait(barrier, 4)

        def load_x(c):
            cp = pltpu.make_async_copy(
                x_ref.at[pl.ds(col0 + c * B, B), :], locb, ldma)
            cp.start()
            return cp

        load_x(my_z).wait()
        commz[0, :, :] = locb[:, :].astype(jnp.bfloat16)
        rcp = pltpu.make_async_copy(
            r_ref.at[pl.ds(col0 + cmine * B, B), :], rbuf, rsem)
        rcp.start()

        for s in range(2):
            rdma = pltpu.make_async_remote_copy(
                src_ref=commz.at[s], dst_ref=commz.at[s + 1],
                send_sem=rs_send.at[s], recv_sem=rs_recv.at[s],
                device_id=(my_x, my_y, nxt),
                device_id_type=pl.DeviceIdType.MESH,
            )
            rdma.start()
            load_x((my_z - s - 1) % N_Z).wait()
            rdma.wait()
            commz[s + 1, :, :] = (commz[s + 1, :, :]
                                  + locb[:, :].astype(jnp.bfloat16))

        zr2 = []
        for h in range(2):
            d = pltpu.make_async_remote_copy(
                src_ref=commz.at[2, pl.ds(h * SUB, SUB), :],
                dst_ref=commz.at[3, pl.ds(h * SUB, SUB), :],
                send_sem=rs2_send.at[h], recv_sem=rs2_recv.at[h],
                device_id=(my_x, my_y, nxt),
                device_id_type=pl.DeviceIdType.MESH,
            )
            d.start()
            zr2.append(d)
        load_x(cmine).wait()
        rcp.wait()

        pending = [None, None]
        ctr = [0]

        def emit_store(val_f32, row0):
            slot = ctr[0] & 1
            if pending[slot] is not None:
                pending[slot].wait()
            ybuf[slot, :, :] = val_f32
            cp = pltpu.make_async_copy(
                ybuf.at[slot], out_ref.at[pl.ds(row0, SUB), :], sdma.at[slot])
            cp.start()
            pending[slot] = cp
            ctr[0] += 1

        sends = []
        zds = {}

        def push_xy(suboff, k):
            for (dst, sd, rv, dev) in ((gx, xs, gxr, xn), (gy, ys, gyr, yn)):
                d = pltpu.make_async_remote_copy(
                    src_ref=colbuf.at[pl.ds(suboff, SUB), :],
                    dst_ref=dst.at[pl.ds(suboff, SUB), :],
                    send_sem=sd.at[k], recv_sem=rv.at[k],
                    device_id=dev, device_id_type=pl.DeviceIdType.MESH,
                )
                d.start()
                sends.append(d)

        for h in range(2):
            zr2[h].wait()
            suboff = cmine * B + h * SUB
            y = (commz[3, pl.ds(h * SUB, SUB), :].astype(jnp.float32)
                 + locb[pl.ds(h * SUB, SUB), :]
                 + rbuf[pl.ds(h * SUB, SUB), :])
            ms = jnp.mean(y * y, axis=1, keepdims=True) + EPS
            o = y * lax.rsqrt(ms) * g_ref[:, :]
            colbuf[pl.ds(suboff, SUB), :] = o.astype(jnp.bfloat16)
            d = pltpu.make_async_remote_copy(
                src_ref=colbuf.at[pl.ds(suboff, SUB), :],
                dst_ref=colbuf.at[pl.ds(suboff, SUB), :],
                send_sem=agz_send.at[h], recv_sem=agz_recv.at[h],
                device_id=(my_x, my_y, nxt),
                device_id_type=pl.DeviceIdType.MESH,
            )
            d.start()
            zds[(0, h)] = d
            push_xy(suboff, h)
            emit_store(o, col0 + suboff)

        for s in range(1, N_Z):
            bs = (my_z - s + 1) % N_Z
            for h in range(2):
                zds[(s - 1, h)].wait()
                suboff = bs * B + h * SUB
                if s < N_Z - 1:
                    k = s * 2 + h
                    d = pltpu.make_async_remote_copy(
                        src_ref=colbuf.at[pl.ds(suboff, SUB), :],
                        dst_ref=colbuf.at[pl.ds(suboff, SUB), :],
                        send_sem=agz_send.at[k], recv_sem=agz_recv.at[k],
                        device_id=(my_x, my_y, nxt),
                        device_id_type=pl.DeviceIdType.MESH,
                    )
                    d.start()
                    zds[(s, h)] = d
                push_xy(suboff, s * 2 + h)
                emit_store(
                    colbuf[pl.ds(suboff, SUB), :].astype(jnp.float32),
                    col0 + suboff)

        qx = 2 * (1 - my_x) + my_y
        qy = 2 * my_x + (1 - my_y)
        qd = 2 * (1 - my_x) + (1 - my_y)

        def recv_wait(buf, suboff, sem):
            d = pltpu.make_async_remote_copy(
                src_ref=buf.at[pl.ds(suboff, SUB), :],
                dst_ref=buf.at[pl.ds(suboff, SUB), :],
                send_sem=sem, recv_sem=sem,
                device_id=xn, device_id_type=pl.DeviceIdType.MESH,
            )
            d.wait_recv()

        for k in range(2 * N_Z):
            s, h = divmod(k, 2)
            suboff = ((my_z - s + 1) % N_Z) * B + h * SUB
            recv_wait(gx, suboff, gxr.at[k])
            if k < 4:
                fwd = pltpu.make_async_remote_copy(
                    src_ref=gx.at[pl.ds(suboff, SUB), :],
                    dst_ref=gd.at[pl.ds(suboff, SUB), :],
                    send_sem=fy.at[k], recv_sem=gdr.at[k],
                    device_id=yn, device_id_type=pl.DeviceIdType.MESH,
                )
                fwd.start()
                sends.append(fwd)
            emit_store(
                gx[pl.ds(suboff, SUB), :].astype(jnp.float32),
                qx * COL + suboff)
            recv_wait(gy, suboff, gyr.at[k])
            if k >= 4:
                fwd = pltpu.make_async_remote_copy(
                    src_ref=gy.at[pl.ds(suboff, SUB), :],
                    dst_ref=gd.at[pl.ds(suboff, SUB), :],
                    send_sem=fx.at[k - 4], recv_sem=gdr.at[k],
                    device_id=xn, device_id_type=pl.DeviceIdType.MESH,
                )
                fwd.start()
                sends.append(fwd)
            emit_store(
                gy[pl.ds(suboff, SUB), :].astype(jnp.float32),
                qy * COL + suboff)

        for k in range(2 * N_Z):
            s, h = divmod(k, 2)
            suboff = ((my_z - s + 1) % N_Z) * B + h * SUB
            recv_wait(gd, suboff, gdr.at[k])
            emit_store(
                gd[pl.ds(suboff, SUB), :].astype(jnp.float32),
                qd * COL + suboff)

        for d in sends:
            d.wait_send()
        for cp in pending:
            if cp is not None:
                cp.wait()

    return pl.pallas_call(
        body,
        out_shape=jax.ShapeDtypeStruct((M, D), jnp.float32),
        in_specs=[
            pl.BlockSpec(memory_space=pl.ANY),
            pl.BlockSpec(memory_space=pl.ANY),
            pl.BlockSpec(memory_space=pltpu.VMEM),
        ],
        out_specs=pl.BlockSpec(memory_space=pl.ANY),
        scratch_shapes=[
            pltpu.VMEM((N_Z, B, D), jnp.bfloat16),
            pltpu.VMEM((B, D), jnp.float32),
            pltpu.VMEM((B, D), jnp.float32),
            pltpu.VMEM((COL, D), jnp.bfloat16),
            pltpu.VMEM((COL, D), jnp.bfloat16),
            pltpu.VMEM((COL, D), jnp.bfloat16),
            pltpu.VMEM((COL, D), jnp.bfloat16),
            pltpu.VMEM((2, SUB, D), jnp.float32),
            pltpu.SemaphoreType.DMA((2,)),
            pltpu.SemaphoreType.DMA((2,)),
            pltpu.SemaphoreType.DMA((2,)),
            pltpu.SemaphoreType.DMA((2,)),
            pltpu.SemaphoreType.DMA((6,)),
            pltpu.SemaphoreType.DMA((6,)),
            pltpu.SemaphoreType.DMA((8,)),
            pltpu.SemaphoreType.DMA((8,)),
            pltpu.SemaphoreType.DMA((8,)),
            pltpu.SemaphoreType.DMA((8,)),
            pltpu.SemaphoreType.DMA((4,)),
            pltpu.SemaphoreType.DMA((4,)),
            pltpu.SemaphoreType.DMA((8,)),
            pltpu.SemaphoreType.DMA,
            pltpu.SemaphoreType.DMA,
            pltpu.SemaphoreType.DMA((2,)),
        ],
        compiler_params=pltpu.CompilerParams(
            collective_id=0,
            vmem_limit_bytes=100 * 1024 * 1024,
        ),
    )(x, r, g)
